# baseline (speedup 1.0000x reference)
import functools

import jax
import jax.numpy as jnp
from jax import lax
from jax.experimental import pallas as pl
from jax.experimental.pallas import tpu as pltpu
from jax.experimental.pallas import tpu_sc as plsc

VOCAB = 100000
HIDDEN = 1024
MAXPOS = 2048
TYPES = 2
EPS = 1e-12
BATCH = 4
SEQ = 2048

N_TOKENS = BATCH * SEQ

NC = 2
NS = 16
NW = NC * NS

SLICE_SEQ = (512, 512, 1024)
SLICE_OFF = tuple(sum(SLICE_SEQ[:i]) for i in range(len(SLICE_SEQ)))
N_SLICES = len(SLICE_SEQ)
CHUNK = 64


def _sc_gather(s, ids_flat, token_table):
    seq_len = SLICE_SEQ[s]
    seq_off = SLICE_OFF[s]
    n_rows = BATCH * seq_len
    tok_per_w = n_rows // NW
    n_chunks = tok_per_w // CHUNK
    nbuf = min(n_chunks, 3, max(1, 393216 // (CHUNK * HIDDEN * 4)))
    mesh = plsc.VectorSubcoreMesh(core_axis_name="c", subcore_axis_name="s")

    @functools.partial(
        pl.kernel,
        mesh=mesh,
        out_type=jax.ShapeDtypeStruct((n_rows, HIDDEN), jnp.float32),
        scratch_types=[
            pltpu.VMEM((tok_per_w,), jnp.int32),
            pltpu.VMEM((nbuf, CHUNK, HIDDEN), jnp.float32),
            pltpu.SemaphoreType.DMA((n_chunks,)),
            pltpu.SemaphoreType.DMA((n_chunks,)),
        ],
    )
    def k(table_hbm, ids_hbm, out_hbm, idx_v, rows_v, gsem, osem):
        wid = lax.axis_index("s") * NC + lax.axis_index("c")
        r = wid * tok_per_w
        b = r // seq_len
        p0 = r % seq_len
        gbase = b * SEQ + seq_off + p0
        pltpu.sync_copy(ids_hbm.at[pl.ds(gbase, tok_per_w)], idx_v)

        def start_gather(c):
            return pltpu.async_copy(
                table_hbm.at[idx_v.at[pl.ds(c * CHUNK, CHUNK)]],
                rows_v.at[c % nbuf], gsem.at[c])

        gathers = [start_gather(c) for c in range(nbuf)]
        outs = []
        for c in range(n_chunks):
            gathers[c].wait()
            o = pltpu.async_copy(
                rows_v.at[c % nbuf], out_hbm.at[pl.ds(r + c * CHUNK, CHUNK)],
                osem.at[c])
            outs.append(o)
            if c + nbuf < n_chunks:
                o.wait()
                gathers.append(start_gather(c + nbuf))
        for c in range(max(n_chunks - nbuf, 0), n_chunks):
            outs[c].wait()

    return k(token_table, ids_flat)


ROWS_BLK = 512
TID_BLOCKS = SEQ // ROWS_BLK


def _tc_epilogue_body(tid_ref, tok_ref, pos_ref, typ_ref, gamma_ref, beta_ref,
                      *rest):
    out_ref = rest[-1]
    x = tok_ref[...] + pos_ref[...]
    te = jnp.where(tid_ref[...] == 0, typ_ref[0:1, :], typ_ref[1:2, :])
    x = x + te
    mean = jnp.mean(x, axis=-1, keepdims=True)
    xc = x - mean
    var = jnp.mean(xc * xc, axis=-1, keepdims=True)
    y = xc * lax.rsqrt(var + EPS)
    out_ref[...] = y * gamma_ref[...] + beta_ref[...]


def _tc_epilogue_slice(s, acc, tok_rows, tid_col, pos_table, type_table,
                       gamma2d, beta2d):
    seq_len = SLICE_SEQ[s]
    seq_off = SLICE_OFF[s]
    nblk = seq_len // ROWS_BLK
    off_blk = seq_off // ROWS_BLK

    in_specs = [
        pl.BlockSpec((ROWS_BLK, 1),
                     lambda i, b: (b * TID_BLOCKS + off_blk + i, 0)),
        pl.BlockSpec((ROWS_BLK, HIDDEN), lambda i, b: (b * nblk + i, 0)),
        pl.BlockSpec((ROWS_BLK, HIDDEN), lambda i, b: (off_blk + i, 0)),
        pl.BlockSpec((TYPES, HIDDEN), lambda i, b: (0, 0)),
        pl.BlockSpec((1, HIDDEN), lambda i, b: (0, 0)),
        pl.BlockSpec((1, HIDDEN), lambda i, b: (0, 0)),
    ]
    args = [tid_col, tok_rows, pos_table, type_table, gamma2d, beta2d]
    io_aliases = {}
    if acc is not None:
        in_specs.append(pl.BlockSpec(memory_space=pl.ANY))
        args.append(acc)
        io_aliases = {6: 0}
    return pl.pallas_call(
        _tc_epilogue_body,
        grid=(nblk, BATCH),
        in_specs=in_specs,
        out_specs=pl.BlockSpec(
            (ROWS_BLK, HIDDEN),
            lambda i, b: (b * TID_BLOCKS + off_blk + i, 0)),
        out_shape=jax.ShapeDtypeStruct((N_TOKENS, HIDDEN), jnp.float32),
        input_output_aliases=io_aliases,
    )(*args)


def kernel(input_ids, token_type_ids, token_table, pos_table, type_table,
           gamma, beta):
    ids_flat = input_ids.reshape(-1).astype(jnp.int32)
    tid_col = token_type_ids.reshape(N_TOKENS, 1).astype(jnp.int32)
    gamma2d = gamma.reshape(1, HIDDEN)
    beta2d = beta.reshape(1, HIDDEN)

    gathered = [_sc_gather(s, ids_flat, token_table)
                for s in range(N_SLICES)]
    acc = None
    for s in range(N_SLICES):
        acc = _tc_epilogue_slice(s, acc, gathered[s], tid_col,
                                 pos_table, type_table, gamma2d, beta2d)
    return acc.reshape(BATCH, SEQ, HIDDEN)


# speedup vs baseline: 1.0194x; 1.0194x over previous
import functools

import jax
import jax.numpy as jnp
from jax import lax
from jax.experimental import pallas as pl
from jax.experimental.pallas import tpu as pltpu
from jax.experimental.pallas import tpu_sc as plsc

VOCAB = 100000
HIDDEN = 1024
MAXPOS = 2048
TYPES = 2
EPS = 1e-12
BATCH = 4
SEQ = 2048

N_TOKENS = BATCH * SEQ

NC = 2
NS = 16
NW = NC * NS

SLICE_SEQ = (512, 1536)
SLICE_OFF = tuple(sum(SLICE_SEQ[:i]) for i in range(len(SLICE_SEQ)))
N_SLICES = len(SLICE_SEQ)
CHUNK = 32


def _sc_gather(s, ids_flat, token_table):
    seq_len = SLICE_SEQ[s]
    seq_off = SLICE_OFF[s]
    n_rows = BATCH * seq_len
    tok_per_w = n_rows // NW
    n_chunks = tok_per_w // CHUNK
    nbuf = min(n_chunks, 3, max(1, 393216 // (CHUNK * HIDDEN * 4)))
    mesh = plsc.VectorSubcoreMesh(core_axis_name="c", subcore_axis_name="s")

    @functools.partial(
        pl.kernel,
        mesh=mesh,
        out_type=jax.ShapeDtypeStruct((n_rows, HIDDEN), jnp.float32),
        scratch_types=[
            pltpu.VMEM((tok_per_w,), jnp.int32),
            pltpu.VMEM((nbuf, CHUNK, HIDDEN), jnp.float32),
            pltpu.SemaphoreType.DMA((n_chunks,)),
            pltpu.SemaphoreType.DMA((n_chunks,)),
        ],
    )
    def k(table_hbm, ids_hbm, out_hbm, idx_v, rows_v, gsem, osem):
        wid = lax.axis_index("s") * NC + lax.axis_index("c")
        r = wid * tok_per_w
        b = r // seq_len
        p0 = r % seq_len
        gbase = b * SEQ + seq_off + p0
        pltpu.sync_copy(ids_hbm.at[pl.ds(gbase, tok_per_w)], idx_v)

        def start_gather(c):
            return pltpu.async_copy(
                table_hbm.at[idx_v.at[pl.ds(c * CHUNK, CHUNK)]],
                rows_v.at[c % nbuf], gsem.at[c])

        gathers = [start_gather(c) for c in range(nbuf)]
        outs = []
        for c in range(n_chunks):
            gathers[c].wait()
            o = pltpu.async_copy(
                rows_v.at[c % nbuf], out_hbm.at[pl.ds(r + c * CHUNK, CHUNK)],
                osem.at[c])
            outs.append(o)
            if c + nbuf < n_chunks:
                o.wait()
                gathers.append(start_gather(c + nbuf))
        for c in range(max(n_chunks - nbuf, 0), n_chunks):
            outs[c].wait()

    return k(token_table, ids_flat)


ROWS_BLK = 512
TID_BLOCKS = SEQ // ROWS_BLK


def _tc_epilogue_body(tid_ref, tok_ref, pos_ref, typ_ref, gamma_ref, beta_ref,
                      *rest):
    out_ref = rest[-1]
    x = tok_ref[...] + pos_ref[...]
    te = jnp.where(tid_ref[...] == 0, typ_ref[0:1, :], typ_ref[1:2, :])
    x = x + te
    mean = jnp.mean(x, axis=-1, keepdims=True)
    xc = x - mean
    var = jnp.mean(xc * xc, axis=-1, keepdims=True)
    y = xc * lax.rsqrt(var + EPS)
    out_ref[...] = y * gamma_ref[...] + beta_ref[...]


def _tc_epilogue_slice(s, acc, tok_rows, tid_col, pos_table, type_table,
                       gamma2d, beta2d):
    seq_len = SLICE_SEQ[s]
    seq_off = SLICE_OFF[s]
    nblk = seq_len // ROWS_BLK
    off_blk = seq_off // ROWS_BLK

    in_specs = [
        pl.BlockSpec((ROWS_BLK, 1),
                     lambda i, b: (b * TID_BLOCKS + off_blk + i, 0)),
        pl.BlockSpec((ROWS_BLK, HIDDEN), lambda i, b: (b * nblk + i, 0)),
        pl.BlockSpec((ROWS_BLK, HIDDEN), lambda i, b: (off_blk + i, 0)),
        pl.BlockSpec((TYPES, HIDDEN), lambda i, b: (0, 0)),
        pl.BlockSpec((1, HIDDEN), lambda i, b: (0, 0)),
        pl.BlockSpec((1, HIDDEN), lambda i, b: (0, 0)),
    ]
    args = [tid_col, tok_rows, pos_table, type_table, gamma2d, beta2d]
    io_aliases = {}
    if acc is not None:
        in_specs.append(pl.BlockSpec(memory_space=pl.ANY))
        args.append(acc)
        io_aliases = {6: 0}
    return pl.pallas_call(
        _tc_epilogue_body,
        grid=(nblk, BATCH),
        in_specs=in_specs,
        out_specs=pl.BlockSpec(
            (ROWS_BLK, HIDDEN),
            lambda i, b: (b * TID_BLOCKS + off_blk + i, 0)),
        out_shape=jax.ShapeDtypeStruct((N_TOKENS, HIDDEN), jnp.float32),
        input_output_aliases=io_aliases,
    )(*args)


def kernel(input_ids, token_type_ids, token_table, pos_table, type_table,
           gamma, beta):
    ids_flat = input_ids.reshape(-1).astype(jnp.int32)
    tid_col = token_type_ids.reshape(N_TOKENS, 1).astype(jnp.int32)
    gamma2d = gamma.reshape(1, HIDDEN)
    beta2d = beta.reshape(1, HIDDEN)

    gathered = [_sc_gather(s, ids_flat, token_table)
                for s in range(N_SLICES)]
    acc = None
    for s in range(N_SLICES):
        acc = _tc_epilogue_slice(s, acc, gathered[s], tid_col,
                                 pos_table, type_table, gamma2d, beta2d)
    return acc.reshape(BATCH, SEQ, HIDDEN)
